# baseline (speedup 1.0000x reference)
import functools
import jax
import jax.numpy as jnp
from jax import lax
from jax.experimental import pallas as pl
from jax.experimental.pallas import tpu as pltpu
from jax.experimental.pallas import tpu_sc as plsc

N = 10000
E = 160000
NC = 2
NS = 16
L = 16
HW = 128

CH = 80
RPT = N // NS

SSW = 25
NSCW = (E // NS) // (SSW * CH)

EP = 163840
SSN = 32
NSCN = (EP // (NC * NS)) // (SSN * CH)
NPAD = EP - E

BN = 1000
GRID = N // BN


def _sc_pipeline(groups, SS, table_v, srcm_slab, dstm_slab, efm_slab, nsc,
                 src_v, dst_v, ef_v, rows_v, wb_v, accum,
                 sg0, sg1, ss0, ss1):
    sgs = (sg0, sg1)
    sss = (ss0, ss1)

    def fire_g(k, b):
        pltpu.async_copy(table_v.at[src_v.at[k]], rows_v.at[b], sgs[b])

    def wait_g(k, b):
        pltpu.make_async_copy(table_v.at[src_v.at[k]], rows_v.at[b],
                              sgs[b]).wait()

    def fire_s(k, b):
        pltpu.async_copy(rows_v.at[b], accum.at[dst_v.at[k]], sss[b],
                         add=True)

    def wait_s(k, b):
        pltpu.make_async_copy(rows_v.at[b], accum.at[dst_v.at[k]],
                              sss[b]).wait()

    def compute(k, b):
        def sub_body(t, _):
            evec = ef_v[k, pl.ds(t * L, L)]
            for j in range(L):
                eb = jnp.full((L,), evec[j], jnp.float32)
                i = t * L + j
                for g in range(groups):
                    sl = pl.ds(g * L, L)
                    rows_v[b, i, sl] = jnp.maximum(
                        rows_v[b, i, sl] + eb * wb_v[sl], 0.0)
            return 0
        lax.fori_loop(0, CH // L, sub_body, 0)

    def super_body(q, _):
        pltpu.sync_copy(srcm_slab(q), src_v)
        pltpu.sync_copy(dstm_slab(q), dst_v)
        pltpu.sync_copy(efm_slab(q), ef_v)

        fire_g(0, 0)

        def pair_body(p, _):
            ka = 2 * p
            kb = ka + 1
            wait_g(ka, 0)

            @pl.when(p > 0)
            def _():
                wait_s(ka - 1, 1)
            fire_g(kb, 1)
            compute(ka, 0)
            fire_s(ka, 0)
            wait_g(kb, 1)
            wait_s(ka, 0)

            @pl.when(ka + 2 < SS)
            def _():
                fire_g(ka + 2, 0)
            compute(kb, 1)
            fire_s(kb, 1)
            return 0
        lax.fori_loop(0, SS // 2, pair_body, 0)

        if SS % 2:
            k = SS - 1
            wait_g(k, 0)
            compute(k, 0)
            fire_s(k, 0)
            wait_s(k, 0)
            wait_s(k - 1, 1)
        else:
            wait_s(SS - 1, 1)
        return 0
    lax.fori_loop(0, nsc, super_body, 0)


def _zero_accum(rows_v, accum, s):
    def zfill(i, _):
        for g in range(HW // L):
            rows_v[0, i, pl.ds(g * L, L)] = jnp.zeros((L,), jnp.float32)
        return 0
    lax.fori_loop(0, CH, zfill, 0)
    for r in range(RPT // CH):
        pltpu.sync_copy(rows_v.at[0], accum.at[pl.ds(s * RPT + r * CH, CH)])
    pltpu.sync_copy(rows_v.at[0].at[pl.ds(0, RPT % CH)],
                    accum.at[pl.ds(s * RPT + (RPT // CH) * CH, RPT % CH)])
    plsc.subcore_barrier()


_MESH = plsc.VectorSubcoreMesh(core_axis_name="c", subcore_axis_name="s")
_SC_SCRATCH = lambda SS, nacc: [
    pltpu.VMEM((SS, CH), jnp.int32),
    pltpu.VMEM((SS, CH), jnp.int32),
    pltpu.VMEM((SS, CH), jnp.float32),
    pltpu.VMEM((2, CH, HW), jnp.float32),
    pltpu.VMEM((HW,), jnp.float32),
    pltpu.VMEM_SHARED((nacc, HW), jnp.float32),
    pltpu.SemaphoreType.DMA,
    pltpu.SemaphoreType.DMA,
    pltpu.SemaphoreType.DMA,
    pltpu.SemaphoreType.DMA,
]


@functools.partial(
    pl.kernel,
    out_type=jax.ShapeDtypeStruct((NC, NS, RPT, HW), jnp.float32),
    mesh=_MESH,
    scratch_types=_SC_SCRATCH(SSW, N),
)
def _sc_msg_wide(table, srcm, dstm, efm, wb, out,
                 src_v, dst_v, ef_v, rows_v, wb_v, accum, sg0, sg1, ss0, ss1):
    c = lax.axis_index("c")
    s = lax.axis_index("s")
    pltpu.sync_copy(wb.at[c], wb_v)
    _zero_accum(rows_v, accum, s)
    _sc_pipeline(8, SSW, table.at[c],
                 lambda q: srcm.at[s, q], lambda q: dstm.at[s, q],
                 lambda q: efm.at[s, q], NSCW,
                 src_v, dst_v, ef_v, rows_v, wb_v, accum,
                 sg0, sg1, ss0, ss1)
    plsc.subcore_barrier()
    pltpu.sync_copy(accum.at[pl.ds(s * RPT, RPT)], out.at[c, s])


@functools.partial(
    pl.kernel,
    out_type=jax.ShapeDtypeStruct((NC, NS, RPT, HW), jnp.float32),
    mesh=_MESH,
    scratch_types=_SC_SCRATCH(SSN, N + 8),
)
def _sc_msg_narrow(table, srcm, dstm, efm, wb, out,
                   src_v, dst_v, ef_v, rows_v, wb_v, accum,
                   sg0, sg1, ss0, ss1):
    c = lax.axis_index("c")
    s = lax.axis_index("s")
    pltpu.sync_copy(wb, wb_v)
    _zero_accum(rows_v, accum, s)
    _sc_pipeline(2, SSN, table,
                 lambda q: srcm.at[c, s, q], lambda q: dstm.at[c, s, q],
                 lambda q: efm.at[c, s, q], NSCN,
                 src_v, dst_v, ef_v, rows_v, wb_v, accum,
                 sg0, sg1, ss0, ss1)
    plsc.subcore_barrier()
    pltpu.sync_copy(accum.at[pl.ds(s * RPT, RPT)], out.at[c, s])


def _pad_cols(x, width):
    bn = x.shape[0]
    return jnp.concatenate(
        [x, jnp.zeros((bn, width - x.shape[1]), jnp.float32)], axis=1)


def _tc1_body(x_ref, w_ref, b_ref, p_ref, q_ref):
    acc = jnp.dot(x_ref[...], w_ref[...],
                  preferred_element_type=jnp.float32) + b_ref[...]
    p_ref[...] = _pad_cols(acc[:, :32], HW)
    q_ref[...] = acc[:, 32:64]


def _tc1(nfeats, wcat, bias):
    return pl.pallas_call(
        _tc1_body,
        grid=(GRID,),
        in_specs=[
            pl.BlockSpec((BN, 256), lambda i: (i, 0)),
            pl.BlockSpec((256, 64), lambda i: (0, 0)),
            pl.BlockSpec((1, 64), lambda i: (0, 0)),
        ],
        out_specs=[
            pl.BlockSpec((BN, HW), lambda i: (i, 0)),
            pl.BlockSpec((BN, 32), lambda i: (i, 0)),
        ],
        out_shape=[
            jax.ShapeDtypeStruct((N, HW), jnp.float32),
            jax.ShapeDtypeStruct((N, 32), jnp.float32),
        ],
    )(nfeats, wcat, bias)


def _make_tc_apply_next(ph, qw):
    nw = 2 * ph + qw
    split_p = ph >= HW

    def body(q_ref, hn_ref, wbt_ref, ba_ref, wn_ref, bn_ref, p_ref, qn_ref):
        hn = hn_ref[0][:, :32] + hn_ref[1][:, :32]
        h = q_ref[...] + jnp.dot(hn, wbt_ref[...],
                                 preferred_element_type=jnp.float32)
        h = jnp.maximum(h + ba_ref[...], 0.0)
        acc = jnp.dot(h, wn_ref[...],
                      preferred_element_type=jnp.float32) + bn_ref[...]
        if split_p:
            p_ref[0, :, :] = acc[:, :ph]
            p_ref[1, :, :] = acc[:, ph:2 * ph]
        else:
            p_ref[...] = _pad_cols(acc[:, :2 * ph], HW)
        qn_ref[...] = acc[:, 2 * ph:]

    p_shape = (jax.ShapeDtypeStruct((NC, N, ph), jnp.float32) if split_p
               else jax.ShapeDtypeStruct((N, HW), jnp.float32))
    p_spec = (pl.BlockSpec((NC, BN, ph), lambda i: (0, i, 0)) if split_p
              else pl.BlockSpec((BN, HW), lambda i: (i, 0)))

    def call(q, hn, wbt, ba, wn, bn):
        return pl.pallas_call(
            body,
            grid=(GRID,),
            in_specs=[
                pl.BlockSpec((BN, 32), lambda i: (i, 0)),
                pl.BlockSpec((NC, BN, HW), lambda i: (0, i, 0)),
                pl.BlockSpec((32, 32), lambda i: (0, 0)),
                pl.BlockSpec((1, 32), lambda i: (0, 0)),
                pl.BlockSpec((32, nw), lambda i: (0, 0)),
                pl.BlockSpec((1, nw), lambda i: (0, 0)),
            ],
            out_specs=[
                p_spec,
                pl.BlockSpec((BN, qw), lambda i: (i, 0)),
            ],
            out_shape=[
                p_shape,
                jax.ShapeDtypeStruct((N, qw), jnp.float32),
            ],
        )(q, hn, wbt, ba, wn, bn)

    return call


_tc2 = _make_tc_apply_next(16, 32)
_tc3 = _make_tc_apply_next(128, 256)


def _tc4_body(r_ref, hn_ref, w_ref, b_ref, o_ref):
    acc = r_ref[...] + b_ref[...]
    acc = acc + jnp.dot(hn_ref[0], w_ref[0],
                        preferred_element_type=jnp.float32)
    acc = acc + jnp.dot(hn_ref[1], w_ref[1],
                        preferred_element_type=jnp.float32)
    acc = jnp.maximum(acc, 0.0)
    o_ref[...] = jnp.sum(acc, axis=1, keepdims=True)


def _tc4(r3, hn3, w3bt, ba3):
    return pl.pallas_call(
        _tc4_body,
        grid=(GRID,),
        in_specs=[
            pl.BlockSpec((BN, 256), lambda i: (i, 0)),
            pl.BlockSpec((NC, BN, HW), lambda i: (0, i, 0)),
            pl.BlockSpec((NC, 128, 256), lambda i: (0, 0, 0)),
            pl.BlockSpec((1, 256), lambda i: (0, 0)),
        ],
        out_specs=pl.BlockSpec((BN, 1), lambda i: (i, 0)),
        out_shape=jax.ShapeDtypeStruct((N, 1), jnp.float32),
    )(r3, hn3, w3bt, ba3)


def kernel(nfeats, efeats, edge_index,
           W_msg1, b_msg1, W_apply1, b_apply1,
           W_msg2, b_msg2, W_apply2, b_apply2,
           W_msg3, b_msg3, W_apply3, b_apply3):
    src = edge_index[0].astype(jnp.int32)
    dst = edge_index[1].astype(jnp.int32)
    ef = efeats.astype(jnp.float32)

    src_w = src.reshape(NS, NSCW, SSW, CH)
    dst_w = dst.reshape(NS, NSCW, SSW, CH)
    ef_w = ef.reshape(NS, NSCW, SSW, CH)

    src_n = jnp.concatenate(
        [src, jnp.zeros((NPAD,), jnp.int32)]).reshape(NC, NS, NSCN, SSN, CH)
    dst_n = jnp.concatenate(
        [dst, jnp.full((NPAD,), N, jnp.int32)]).reshape(NC, NS, NSCN, SSN, CH)
    ef_n = jnp.concatenate(
        [ef, jnp.zeros((NPAD,), jnp.float32)]).reshape(NC, NS, NSCN, SSN, CH)

    w1cat = jnp.concatenate([W_msg1[:, :256].T, W_apply1[:, :256].T], axis=1)
    b1cat = jnp.concatenate([b_msg1, jnp.zeros((32,), jnp.float32)])[None, :]
    w1b = _pad_cols(W_msg1[:, 256][None, :], HW)[0]

    w1bt = W_apply1[:, 256:].T
    ba1 = b_apply1[None, :]
    w2cat = jnp.concatenate([W_msg2[:, :32].T, W_apply2[:, :32].T], axis=1)
    b2cat = jnp.concatenate([b_msg2, jnp.zeros((32,), jnp.float32)])[None, :]
    w2b = _pad_cols(W_msg2[:, 32][None, :], HW)[0]

    w2bt = W_apply2[:, 32:].T
    ba2 = b_apply2[None, :]
    w3cat = jnp.concatenate([W_msg3[:, :32].T, W_apply3[:, :32].T], axis=1)
    b3cat = jnp.concatenate([b_msg3, jnp.zeros((256,), jnp.float32)])[None, :]
    w3b = W_msg3[:, 32].reshape(NC, 128)

    w3bt = jnp.stack([W_apply3[:, 32:].T[:128], W_apply3[:, 32:].T[128:]])
    ba3 = b_apply3[None, :]

    p1, q1 = _tc1(nfeats, w1cat, b1cat)
    hn1 = _sc_msg_narrow(p1, src_n, dst_n, ef_n, w1b).reshape(NC, N, HW)
    p2, q2 = _tc2(q1, hn1, w1bt, ba1, w2cat, b2cat)
    hn2 = _sc_msg_narrow(p2, src_n, dst_n, ef_n, w2b).reshape(NC, N, HW)
    p3, r3 = _tc3(q2, hn2, w2bt, ba2, w3cat, b3cat)
    hn3 = _sc_msg_wide(p3, src_w, dst_w, ef_w, w3b).reshape(NC, N, HW)
    out = _tc4(r3, hn3, w3bt, ba3)
    return out.reshape(N)
